# Initial kernel scaffold; baseline (speedup 1.0000x reference)
import jax
import jax.numpy as jnp
from jax.experimental import pallas as pl


def kernel(input_ids, pos_table):
    raise NotImplementedError("write your pallas kernel here")


# speedup vs baseline: 2.7460x; 2.7460x over previous
import jax
import jax.numpy as jnp
from jax.experimental import pallas as pl

_BLOCK_ROWS = 512


def _copy_body(src_ref, out_ref):
    out_ref[...] = src_ref[...]


def kernel(input_ids, pos_table):
    seq_len = input_ids.shape[1]
    rows, dim = pos_table.shape
    assert seq_len == rows
    grid = (rows // _BLOCK_ROWS,)
    return pl.pallas_call(
        _copy_body,
        grid=grid,
        in_specs=[pl.BlockSpec((_BLOCK_ROWS, dim), lambda i: (i, 0))],
        out_specs=pl.BlockSpec((_BLOCK_ROWS, dim), lambda i: (i, 0)),
        out_shape=jax.ShapeDtypeStruct((seq_len, dim), pos_table.dtype),
    )(pos_table)


# speedup vs baseline: 2.9740x; 1.0830x over previous
import jax
import jax.numpy as jnp
from jax.experimental import pallas as pl

_BLOCK_ROWS = 1024


def _copy_body(src_ref, out_ref):
    out_ref[...] = src_ref[...]


def kernel(input_ids, pos_table):
    seq_len = input_ids.shape[1]
    rows, dim = pos_table.shape
    assert seq_len == rows
    grid = (rows // _BLOCK_ROWS,)
    return pl.pallas_call(
        _copy_body,
        grid=grid,
        in_specs=[pl.BlockSpec((_BLOCK_ROWS, dim), lambda i: (i, 0))],
        out_specs=pl.BlockSpec((_BLOCK_ROWS, dim), lambda i: (i, 0)),
        out_shape=jax.ShapeDtypeStruct((seq_len, dim), pos_table.dtype),
    )(pos_table)


# speedup vs baseline: 3.1750x; 1.0676x over previous
import jax
import jax.numpy as jnp
from jax.experimental import pallas as pl

_BLOCK_ROWS = 2048


def _copy_body(src_ref, out_ref):
    out_ref[...] = src_ref[...]


def kernel(input_ids, pos_table):
    seq_len = input_ids.shape[1]
    rows, dim = pos_table.shape
    assert seq_len == rows
    grid = (rows // _BLOCK_ROWS,)
    return pl.pallas_call(
        _copy_body,
        grid=grid,
        in_specs=[pl.BlockSpec((_BLOCK_ROWS, dim), lambda i: (i, 0))],
        out_specs=pl.BlockSpec((_BLOCK_ROWS, dim), lambda i: (i, 0)),
        out_shape=jax.ShapeDtypeStruct((seq_len, dim), pos_table.dtype),
    )(pos_table)
